# baseline (speedup 1.0000x reference)
import jax, jax.numpy as jnp
from jax import lax
from jax.experimental import pallas as pl

_IOU = 0.7
_SCORE_T = 0.01
_TOPK = 300


def _iou_one_many(box, boxes):
    lt = jnp.maximum(box[:2], boxes[:, :2])
    rb = jnp.minimum(box[2:], boxes[:, 2:])
    wh = jnp.clip(rb - lt, 0.0, None)
    inter = wh[:, 0] * wh[:, 1]
    area1 = (box[2] - box[0]) * (box[3] - box[1])
    area2 = (boxes[:, 2] - boxes[:, 0]) * (boxes[:, 3] - boxes[:, 1])
    return inter / (area1 + area2 - inter + 1e-9)


def _nms_one(boxes, scores, labels):
    s0 = jnp.where(scores > _SCORE_T, scores, -jnp.inf)
    max_coord = jnp.max(boxes) + 1.0
    b = boxes + (labels.astype(boxes.dtype) * max_coord)[:, None]

    def step(s, _):
        idx = jnp.argmax(s)
        ok = s[idx] > -jnp.inf
        kidx = jnp.where(ok, idx.astype(jnp.int32), jnp.int32(-1))
        ious = _iou_one_many(b[idx], b)
        s = jnp.where(ious > _IOU, -jnp.inf, s)
        s = s.at[idx].set(-jnp.inf)
        return s, kidx

    _, keep = lax.scan(step, s0, None, length=_TOPK)
    safe = jnp.maximum(keep, 0)
    m = keep >= 0
    out_boxes = boxes[safe] * m[:, None].astype(boxes.dtype)
    out_scores = scores[safe] * m.astype(scores.dtype)
    out_labels = jnp.where(m, labels[safe], -1)
    return out_labels, out_boxes, out_scores


def kernel(pred_logits, pred_boxes, orig_target_sizes):
    cx = pred_boxes[..., 0:1]
    cy = pred_boxes[..., 1:2]
    w = pred_boxes[..., 2:3]
    h = pred_boxes[..., 3:4]
    xyxy = jnp.concatenate([cx - 0.5 * w, cy - 0.5 * h, cx + 0.5 * w, cy + 0.5 * h], axis=-1)
    scale = jnp.tile(orig_target_sizes.astype(jnp.float32), (1, 2))[:, None, :]
    boxes = xyxy * scale
    values = jnp.max(pred_logits, axis=-1)
    labels = jnp.argmax(pred_logits, axis=-1).astype(jnp.int32)
    scores = jax.nn.sigmoid(values)
    return jax.vmap(_nms_one)(boxes, scores, labels)


# speedup vs baseline: 24.2216x; 24.2216x over previous
import functools

import jax
import jax.numpy as jnp
from jax import lax
from jax.experimental import pallas as pl
from jax.experimental.pallas import tpu as pltpu
from jax.experimental.pallas import tpu_sc as plsc

_B, _N, _C = 8, 5000, 91
_NPAD = 5120
_NG = 20
_KPAD = 304
_TOPK = 300
_IOU_T = 0.7
_SCORE_T = 0.01
_NROW = 16


def _prep_body(lg_ref, bx_ref, sc_ref, out_ref):
    i = pl.program_id(0)
    lg = lg_ref[0]
    v = jnp.max(lg, axis=0, keepdims=True)
    cio = lax.broadcasted_iota(jnp.int32, (_C, _NPAD), 0)
    lab = jnp.min(jnp.where(lg == v, cio, _C), axis=0, keepdims=True)
    labf = lab.astype(jnp.float32)

    cx = bx_ref[0, 0:1, :]
    cy = bx_ref[0, 1:2, :]
    w = bx_ref[0, 2:3, :]
    h = bx_ref[0, 3:4, :]
    sw = sc_ref[i, 0]
    sh = sc_ref[i, 1]
    x1 = (cx - 0.5 * w) * sw
    y1 = (cy - 0.5 * h) * sh
    x2 = (cx + 0.5 * w) * sw
    y2 = (cy + 0.5 * h) * sh
    m = jnp.maximum(jnp.maximum(jnp.max(x1), jnp.max(y1)),
                    jnp.maximum(jnp.max(x2), jnp.max(y2))) + 1.0
    t = labf * m
    out_ref[0, 0:1, :] = v
    out_ref[0, 1:2, :] = labf
    out_ref[0, 2:3, :] = x1 + t
    out_ref[0, 3:4, :] = y1 + t
    out_ref[0, 4:5, :] = x2 + t
    out_ref[0, 5:6, :] = y2 + t
    out_ref[0, 6:7, :] = x1
    out_ref[0, 7:8, :] = y1
    out_ref[0, 8:9, :] = x2
    out_ref[0, 9:10, :] = y2
    out_ref[0, 10:_NROW, :] = jnp.zeros((_NROW - 10, _NPAD), jnp.float32)


def _prep(logits_t, boxes_t, scale):
    return pl.pallas_call(
        _prep_body,
        grid=(_B,),
        in_specs=[
            pl.BlockSpec((1, _C, _NPAD), lambda i: (i, 0, 0)),
            pl.BlockSpec((1, 4, _NPAD), lambda i: (i, 0, 0)),
            pl.BlockSpec(memory_space=pltpu.SMEM),
        ],
        out_specs=pl.BlockSpec((1, _NROW, _NPAD), lambda i: (i, 0, 0)),
        out_shape=jax.ShapeDtypeStruct((_B, _NROW, _NPAD), jnp.float32),
    )(logits_t, boxes_t, scale)


def _nms_image(img, s_hbm, st_hbm,
               outl_h, ob1_h, ob2_h, ob3_h, ob4_h, osc_h,
               s0v, lblv, bx1, by1, bx2, by2, ox1, oy1, ox2, oy2, bm,
               kx1, ky1, kx2, ky2, outl, outb1, outb2, outb3, outb4, outs,
               sem):
        copies = [
            pltpu.async_copy(s_hbm.at[img], s0v, sem),
            pltpu.async_copy(st_hbm.at[img, 1], lblv, sem),
            pltpu.async_copy(st_hbm.at[img, 2], bx1, sem),
            pltpu.async_copy(st_hbm.at[img, 3], by1, sem),
            pltpu.async_copy(st_hbm.at[img, 4], bx2, sem),
            pltpu.async_copy(st_hbm.at[img, 5], by2, sem),
            pltpu.async_copy(st_hbm.at[img, 6], ox1, sem),
            pltpu.async_copy(st_hbm.at[img, 7], oy1, sem),
            pltpu.async_copy(st_hbm.at[img, 8], ox2, sem),
            pltpu.async_copy(st_hbm.at[img, 9], oy2, sem),
        ]
        for cp in copies:
            cp.wait()

        iota = lax.iota(jnp.int32, 16)
        ninf = jnp.full((16,), -jnp.inf, jnp.float32)

        def initg(g, c):
            def initi(i, mcar):
                off = g * 256 + i * 16
                vv = s0v[pl.ds(off, 16)]
                vv = jnp.where(vv > _SCORE_T, vv, -jnp.inf)
                s0v[pl.ds(off, 16)] = vv
                return jnp.maximum(mcar, vv)
            mg = lax.fori_loop(0, 16, initi, ninf)
            bm[pl.ds(g * 16, 16)] = mg
            return c
        lax.fori_loop(0, _NG, initg, 0)

        def initk(t, c):
            off = t * 16
            big = jnp.full((16,), 1e30, jnp.float32)
            z = jnp.zeros((16,), jnp.float32)
            kx1[pl.ds(off, 16)] = big
            ky1[pl.ds(off, 16)] = big
            kx2[pl.ds(off, 16)] = big
            ky2[pl.ds(off, 16)] = big
            outl[pl.ds(off, 16)] = jnp.full((16,), -1.0, jnp.float32)
            outb1[pl.ds(off, 16)] = z
            outb2[pl.ds(off, 16)] = z
            outb3[pl.ds(off, 16)] = z
            outb4[pl.ds(off, 16)] = z
            outs[pl.ds(off, 16)] = z
            return c
        lax.fori_loop(0, _KPAD // 16, initk, 0)

        bigi = jnp.int32(2 ** 30)

        def attempt(_, carry):
            nk, alive = carry
            inb = nk < _TOPK

            def p1(g, mcar):
                return jnp.maximum(mcar, bm[pl.ds(g * 16, 16)])
            mv = lax.fori_loop(0, _NG, p1, ninf)
            gm = jnp.max(mv)
            alive2 = jnp.where(gm > -jnp.inf, alive, jnp.int32(0))

            def p2a(g, gcur):
                bmv = bm[pl.ds(g * 16, 16)]
                hasv = jnp.max(jnp.where(bmv == gm, 1, 0))
                return jnp.where((hasv > 0) & (g < gcur), g, gcur)
            gstar = lax.fori_loop(0, _NG, p2a, jnp.int32(_NG))
            gstar = jnp.where(gstar >= _NG, 0, gstar)
            gbase = gstar * 256

            def p2b(i, jmc):
                off = gbase + i * 16
                vv = s0v[pl.ds(off, 16)]
                return jnp.minimum(jmc, jnp.where(vv == gm, off + iota, bigi))
            jm = lax.fori_loop(0, 16, p2b, jnp.full((16,), bigi, jnp.int32))
            j = jnp.min(jm)
            j = jnp.where(alive2 == 1, j, 0)

            jbase = (j // 16) * 16
            jmsk = iota == (j - jbase)

            def pick(ref):
                return jnp.sum(jnp.where(jmsk, ref[pl.ds(jbase, 16)], 0.0))

            wx1 = pick(bx1)
            wy1 = pick(by1)
            wx2 = pick(bx2)
            wy2 = pick(by2)
            carea = (wx2 - wx1) * (wy2 - wy1)

            nv = (nk + 15) // 16

            def vb(t, acc):
                off = t * 16
                a1 = kx1[pl.ds(off, 16)]
                c1 = ky1[pl.ds(off, 16)]
                a2 = kx2[pl.ds(off, 16)]
                c2 = ky2[pl.ds(off, 16)]
                ltx = jnp.maximum(a1, wx1)
                lty = jnp.maximum(c1, wy1)
                rbx = jnp.minimum(a2, wx2)
                rby = jnp.minimum(c2, wy2)
                whx = jnp.maximum(rbx - ltx, 0.0)
                why = jnp.maximum(rby - lty, 0.0)
                inter = whx * why
                karea = (a2 - a1) * (c2 - c1)
                denom = (karea + carea) - inter + 1e-9
                iou = inter / denom
                return jnp.maximum(acc, jnp.where(iou > _IOU_T, 1, 0))
            acc = lax.fori_loop(0, nv, vb, jnp.zeros((16,), jnp.int32))
            sup = jnp.max(acc)
            keepf = (alive2 == 1) & (sup == 0) & inb

            @pl.when((alive2 == 1) & inb)
            def _():
                base = (j // 16) * 16
                lane = j - base
                vv = s0v[pl.ds(base, 16)]
                s0v[pl.ds(base, 16)] = jnp.where(iota == lane, -jnp.inf, vv)
                g = j // 256

                def upd(i, mcar):
                    return jnp.maximum(mcar, s0v[pl.ds(g * 256 + i * 16, 16)])
                bm[pl.ds(g * 16, 16)] = lax.fori_loop(0, 16, upd, ninf)

            @pl.when(keepf)
            def _():
                base = (nk // 16) * 16
                lane = nk - base
                lm = iota == lane

                def put(ref, val):
                    old = ref[pl.ds(base, 16)]
                    ref[pl.ds(base, 16)] = jnp.where(lm, val, old)
                put(kx1, wx1)
                put(ky1, wy1)
                put(kx2, wx2)
                put(ky2, wy2)
                put(outl, pick(lblv))
                put(outb1, pick(ox1))
                put(outb2, pick(oy1))
                put(outb3, pick(ox2))
                put(outb4, pick(oy2))
                put(outs, gm)

            nk2 = nk + jnp.where(keepf, jnp.int32(1), jnp.int32(0))
            return (nk2, alive2)

        def chunk(_, carry):
            nk, alive = carry
            active = (nk < _TOPK) & (alive == 1)
            return lax.cond(
                active,
                lambda c: lax.fori_loop(0, 16, attempt, c),
                lambda c: c,
                carry)

        lax.fori_loop(0, 339, chunk, (jnp.int32(0), jnp.int32(1)))

        pltpu.sync_copy(outl, outl_h.at[img])
        pltpu.sync_copy(outb1, ob1_h.at[img])
        pltpu.sync_copy(outb2, ob2_h.at[img])
        pltpu.sync_copy(outb3, ob3_h.at[img])
        pltpu.sync_copy(outb4, ob4_h.at[img])
        pltpu.sync_copy(outs, osc_h.at[img])


def _nms_body(*args):
    wid = lax.axis_index("s") * 2 + lax.axis_index("c")

    @pl.when(wid < _B)
    def _():
        _nms_image(wid, *args)


_nms_call = None


def _nms(*args):
    global _nms_call
    if _nms_call is None:
        _nms_call = functools.partial(
            pl.kernel,
            out_type=[jax.ShapeDtypeStruct((_B, _KPAD), jnp.float32)] * 6,
            mesh=plsc.VectorSubcoreMesh(
                core_axis_name="c", subcore_axis_name="s",
                num_cores=2, num_subcores=16),
            compiler_params=pltpu.CompilerParams(needs_layout_passes=False),
            scratch_types=(
                [pltpu.VMEM((_NPAD,), jnp.float32)] * 10
                + [pltpu.VMEM((_NG * 16,), jnp.float32)]
                + [pltpu.VMEM((_KPAD,), jnp.float32)] * 10
                + [pltpu.SemaphoreType.DMA]
            ),
        )(_nms_body)
    return _nms_call(*args)


def kernel(pred_logits, pred_boxes, orig_target_sizes):
    f32 = jnp.float32
    lg = jnp.transpose(pred_logits, (0, 2, 1))
    lg = jnp.pad(lg, ((0, 0), (0, 0), (0, _NPAD - _N)), constant_values=-1e30)
    bx = jnp.transpose(pred_boxes, (0, 2, 1))
    bx = jnp.pad(bx, ((0, 0), (0, 0), (0, _NPAD - _N)))
    scale = jnp.tile(orig_target_sizes.astype(f32), (1, 2))

    stacked = _prep(lg, bx, scale)
    s = jax.nn.sigmoid(stacked[:, 0, :])
    outl, ob1, ob2, ob3, ob4, osc = _nms(s, stacked)

    out_labels = outl[:, :_TOPK].astype(jnp.int32)
    out_boxes = jnp.stack(
        [ob1[:, :_TOPK], ob2[:, :_TOPK], ob3[:, :_TOPK], ob4[:, :_TOPK]],
        axis=-1)
    out_scores = osc[:, :_TOPK]
    return out_labels, out_boxes, out_scores


# speedup vs baseline: 30.1839x; 1.2462x over previous
import functools

import jax
import jax.numpy as jnp
from jax import lax
from jax.experimental import pallas as pl
from jax.experimental.pallas import tpu as pltpu
from jax.experimental.pallas import tpu_sc as plsc

_B, _N, _C = 8, 5000, 91
_NPAD = 5120
_NG = 20
_KPAD = 304
_TOPK = 300
_IOU_T = 0.7
_SCORE_T = 0.01
_NROW = 16


def _prep_body(lg_ref, bx_ref, sc_ref, out_ref):
    i = pl.program_id(0)
    lg = lg_ref[0]
    v = jnp.max(lg, axis=0, keepdims=True)
    cio = lax.broadcasted_iota(jnp.int32, (_C, _NPAD), 0)
    lab = jnp.min(jnp.where(lg == v, cio, _C), axis=0, keepdims=True)
    labf = lab.astype(jnp.float32)

    cx = bx_ref[0, 0:1, :]
    cy = bx_ref[0, 1:2, :]
    w = bx_ref[0, 2:3, :]
    h = bx_ref[0, 3:4, :]
    sw = sc_ref[i, 0]
    sh = sc_ref[i, 1]
    x1 = (cx - 0.5 * w) * sw
    y1 = (cy - 0.5 * h) * sh
    x2 = (cx + 0.5 * w) * sw
    y2 = (cy + 0.5 * h) * sh
    m = jnp.maximum(jnp.maximum(jnp.max(x1), jnp.max(y1)),
                    jnp.maximum(jnp.max(x2), jnp.max(y2))) + 1.0
    t = labf * m
    out_ref[0, 0:1, :] = v
    out_ref[0, 1:2, :] = labf
    out_ref[0, 2:3, :] = x1 + t
    out_ref[0, 3:4, :] = y1 + t
    out_ref[0, 4:5, :] = x2 + t
    out_ref[0, 5:6, :] = y2 + t
    out_ref[0, 6:7, :] = x1
    out_ref[0, 7:8, :] = y1
    out_ref[0, 8:9, :] = x2
    out_ref[0, 9:10, :] = y2
    out_ref[0, 10:_NROW, :] = jnp.zeros((_NROW - 10, _NPAD), jnp.float32)


def _prep(logits_t, boxes_t, scale):
    return pl.pallas_call(
        _prep_body,
        grid=(_B,),
        in_specs=[
            pl.BlockSpec((1, _C, _NPAD), lambda i: (i, 0, 0)),
            pl.BlockSpec((1, 4, _NPAD), lambda i: (i, 0, 0)),
            pl.BlockSpec(memory_space=pltpu.SMEM),
        ],
        out_specs=pl.BlockSpec((1, _NROW, _NPAD), lambda i: (i, 0, 0)),
        out_shape=jax.ShapeDtypeStruct((_B, _NROW, _NPAD), jnp.float32),
    )(logits_t, boxes_t, scale)


def _nms_image(img, s_hbm, st_hbm,
               outl_h, ob1_h, ob2_h, ob3_h, ob4_h, osc_h,
               s0v, lblv, bx1, by1, bx2, by2, ox1, oy1, ox2, oy2, bm,
               kx1, ky1, kx2, ky2, outl, outb1, outb2, outb3, outb4, outs,
               sem):
        copies = [
            pltpu.async_copy(s_hbm.at[img], s0v, sem),
            pltpu.async_copy(st_hbm.at[img, 1], lblv, sem),
            pltpu.async_copy(st_hbm.at[img, 2], bx1, sem),
            pltpu.async_copy(st_hbm.at[img, 3], by1, sem),
            pltpu.async_copy(st_hbm.at[img, 4], bx2, sem),
            pltpu.async_copy(st_hbm.at[img, 5], by2, sem),
            pltpu.async_copy(st_hbm.at[img, 6], ox1, sem),
            pltpu.async_copy(st_hbm.at[img, 7], oy1, sem),
            pltpu.async_copy(st_hbm.at[img, 8], ox2, sem),
            pltpu.async_copy(st_hbm.at[img, 9], oy2, sem),
        ]
        for cp in copies:
            cp.wait()

        iota = lax.iota(jnp.int32, 16)
        ninf = jnp.full((16,), -jnp.inf, jnp.float32)

        def initg(g, c):
            def initi(i, mcar):
                off = g * 256 + i * 16
                vv = s0v[pl.ds(off, 16)]
                vv = jnp.where(vv > _SCORE_T, vv, -jnp.inf)
                s0v[pl.ds(off, 16)] = vv
                return jnp.maximum(mcar, vv)
            mg = lax.fori_loop(0, 16, initi, ninf)
            bm[pl.ds(g * 16, 16)] = mg
            return c
        lax.fori_loop(0, _NG, initg, 0)

        def initk(t, c):
            off = t * 16
            big = jnp.full((16,), 1e30, jnp.float32)
            z = jnp.zeros((16,), jnp.float32)
            kx1[pl.ds(off, 16)] = big
            ky1[pl.ds(off, 16)] = big
            kx2[pl.ds(off, 16)] = big
            ky2[pl.ds(off, 16)] = big
            outl[pl.ds(off, 16)] = jnp.full((16,), -1.0, jnp.float32)
            outb1[pl.ds(off, 16)] = z
            outb2[pl.ds(off, 16)] = z
            outb3[pl.ds(off, 16)] = z
            outb4[pl.ds(off, 16)] = z
            outs[pl.ds(off, 16)] = z
            return c
        lax.fori_loop(0, _KPAD // 16, initk, 0)

        bigi = jnp.int32(2 ** 30)

        def _tree(vals, op):
            while len(vals) > 1:
                nxt = [op(vals[k], vals[k + 1])
                       for k in range(0, len(vals) - 1, 2)]
                if len(vals) % 2:
                    nxt.append(vals[-1])
                vals = nxt
            return vals[0]

        def attempt(_, carry):
            nk, alive = carry
            inb = nk < _TOPK

            rows = [bm[pl.ds(g * 16, 16)] for g in range(_NG)]
            mv = _tree(rows, jnp.maximum)
            gm = jnp.max(mv)
            alive2 = jnp.where(gm > -jnp.inf, alive, jnp.int32(0))

            gv = _tree([jnp.where(rows[g] == gm, jnp.int32(g), jnp.int32(_NG))
                        for g in range(_NG)], jnp.minimum)
            gstar = jnp.min(gv)
            gstar = jnp.where(gstar >= _NG, 0, gstar)
            gbase = gstar * 256

            cands = []
            for i in range(16):
                off = gbase + i * 16
                vv = s0v[pl.ds(off, 16)]
                cands.append(jnp.where(vv == gm, off + iota, bigi))
            jm = _tree(cands, jnp.minimum)
            j = jnp.min(jm)
            j = jnp.where(alive2 == 1, j, 0)

            jbase = (j // 16) * 16
            jmsk = iota == (j - jbase)

            def pick(ref):
                return jnp.sum(jnp.where(jmsk, ref[pl.ds(jbase, 16)], 0.0))

            wx1 = pick(bx1)
            wy1 = pick(by1)
            wx2 = pick(bx2)
            wy2 = pick(by2)
            carea = (wx2 - wx1) * (wy2 - wy1)

            ious = []
            for t in range(_KPAD // 16):
                off = t * 16
                a1 = kx1[pl.ds(off, 16)]
                c1 = ky1[pl.ds(off, 16)]
                a2 = kx2[pl.ds(off, 16)]
                c2 = ky2[pl.ds(off, 16)]
                ltx = jnp.maximum(a1, wx1)
                lty = jnp.maximum(c1, wy1)
                rbx = jnp.minimum(a2, wx2)
                rby = jnp.minimum(c2, wy2)
                whx = jnp.maximum(rbx - ltx, 0.0)
                why = jnp.maximum(rby - lty, 0.0)
                inter = whx * why
                karea = (a2 - a1) * (c2 - c1)
                denom = (karea + carea) - inter + 1e-9
                ious.append(inter / denom)
            ioumax = jnp.max(_tree(ious, jnp.maximum))
            keepf = (alive2 == 1) & jnp.logical_not(ioumax > _IOU_T) & inb

            @pl.when((alive2 == 1) & inb)
            def _():
                base = (j // 16) * 16
                lane = j - base
                vv = s0v[pl.ds(base, 16)]
                s0v[pl.ds(base, 16)] = jnp.where(iota == lane, -jnp.inf, vv)
                g = j // 256
                grows = [s0v[pl.ds(g * 256 + i * 16, 16)] for i in range(16)]
                bm[pl.ds(g * 16, 16)] = _tree(grows, jnp.maximum)

            @pl.when(keepf)
            def _():
                base = (nk // 16) * 16
                lane = nk - base
                lm = iota == lane

                def put(ref, val):
                    old = ref[pl.ds(base, 16)]
                    ref[pl.ds(base, 16)] = jnp.where(lm, val, old)
                put(kx1, wx1)
                put(ky1, wy1)
                put(kx2, wx2)
                put(ky2, wy2)
                put(outl, pick(lblv))
                put(outb1, pick(ox1))
                put(outb2, pick(oy1))
                put(outb3, pick(ox2))
                put(outb4, pick(oy2))
                put(outs, gm)

            nk2 = nk + jnp.where(keepf, jnp.int32(1), jnp.int32(0))
            return (nk2, alive2)

        def chunk(_, carry):
            nk, alive = carry
            active = (nk < _TOPK) & (alive == 1)
            return lax.cond(
                active,
                lambda c: lax.fori_loop(0, 16, attempt, c),
                lambda c: c,
                carry)

        lax.fori_loop(0, 339, chunk, (jnp.int32(0), jnp.int32(1)))

        pltpu.sync_copy(outl, outl_h.at[img])
        pltpu.sync_copy(outb1, ob1_h.at[img])
        pltpu.sync_copy(outb2, ob2_h.at[img])
        pltpu.sync_copy(outb3, ob3_h.at[img])
        pltpu.sync_copy(outb4, ob4_h.at[img])
        pltpu.sync_copy(outs, osc_h.at[img])


def _nms_body(*args):
    wid = lax.axis_index("s") * 2 + lax.axis_index("c")

    @pl.when(wid < _B)
    def _():
        _nms_image(wid, *args)


_nms_call = None


def _nms(*args):
    global _nms_call
    if _nms_call is None:
        _nms_call = functools.partial(
            pl.kernel,
            out_type=[jax.ShapeDtypeStruct((_B, _KPAD), jnp.float32)] * 6,
            mesh=plsc.VectorSubcoreMesh(
                core_axis_name="c", subcore_axis_name="s",
                num_cores=2, num_subcores=16),
            compiler_params=pltpu.CompilerParams(needs_layout_passes=False),
            scratch_types=(
                [pltpu.VMEM((_NPAD,), jnp.float32)] * 10
                + [pltpu.VMEM((_NG * 16,), jnp.float32)]
                + [pltpu.VMEM((_KPAD,), jnp.float32)] * 10
                + [pltpu.SemaphoreType.DMA]
            ),
        )(_nms_body)
    return _nms_call(*args)


def kernel(pred_logits, pred_boxes, orig_target_sizes):
    f32 = jnp.float32
    lg = jnp.transpose(pred_logits, (0, 2, 1))
    lg = jnp.pad(lg, ((0, 0), (0, 0), (0, _NPAD - _N)), constant_values=-1e30)
    bx = jnp.transpose(pred_boxes, (0, 2, 1))
    bx = jnp.pad(bx, ((0, 0), (0, 0), (0, _NPAD - _N)))
    scale = jnp.tile(orig_target_sizes.astype(f32), (1, 2))

    stacked = _prep(lg, bx, scale)
    s = jax.nn.sigmoid(stacked[:, 0, :])
    outl, ob1, ob2, ob3, ob4, osc = _nms(s, stacked)

    out_labels = outl[:, :_TOPK].astype(jnp.int32)
    out_boxes = jnp.stack(
        [ob1[:, :_TOPK], ob2[:, :_TOPK], ob3[:, :_TOPK], ob4[:, :_TOPK]],
        axis=-1)
    out_scores = osc[:, :_TOPK]
    return out_labels, out_boxes, out_scores
